# Initial kernel scaffold; baseline (speedup 1.0000x reference)
import jax
import jax.numpy as jnp
from jax.experimental import pallas as pl


def kernel(x, edge_index, W1, a_src1, a_dst1, b1, W2, a_src2, a_dst2, b2):
    raise NotImplementedError("write your pallas kernel here")


# speedup vs baseline: 1.1181x; 1.1181x over previous
import functools

import jax
import jax.numpy as jnp
from jax.experimental import pallas as pl
from jax.experimental.pallas import tpu as pltpu

N = 10000
E = 160000
HEADS = 4


def _mm_kernel(x_ref, w_ref, o_ref):
    o_ref[...] = jnp.dot(x_ref[...], w_ref[...],
                         preferred_element_type=jnp.float32)


def _matmul(x, w, blk_r=1024):
    n, k = x.shape
    _, m = w.shape
    n_pad = ((n + blk_r - 1) // blk_r) * blk_r
    if n_pad != n:
        x = jnp.pad(x, ((0, n_pad - n), (0, 0)))
    out = pl.pallas_call(
        _mm_kernel,
        grid=(n_pad // blk_r,),
        in_specs=[
            pl.BlockSpec((blk_r, k), lambda i: (i, 0)),
            pl.BlockSpec((k, m), lambda i: (0, 0)),
        ],
        out_specs=pl.BlockSpec((blk_r, m), lambda i: (i, 0)),
        out_shape=jax.ShapeDtypeStruct((n_pad, m), jnp.float32),
    )(x, w)
    return out[:n]


def _gat_layer(x, src, dst, W, a_src, a_dst, bias, heads, out_c, concat):
    n = x.shape[0]
    h = _matmul(x, W).reshape(n, heads, out_c)
    alpha_src = (h * a_src[None, :, :]).sum(-1)
    alpha_dst = (h * a_dst[None, :, :]).sum(-1)
    alpha = alpha_src[src] + alpha_dst[dst]
    alpha = jax.nn.leaky_relu(alpha, negative_slope=0.2)
    ex = jnp.exp(alpha)
    denom = jax.ops.segment_sum(ex, dst, num_segments=n)
    msg = h[src] * ex[:, :, None]
    out = jax.ops.segment_sum(msg, dst, num_segments=n)
    out = out / (denom[:, :, None] + 1e-16)
    if concat:
        out = out.reshape(n, heads * out_c)
    else:
        out = out.mean(axis=1)
    return out + bias


def kernel(x, edge_index, W1, a_src1, a_dst1, b1, W2, a_src2, a_dst2, b2):
    n = x.shape[0]
    loop = jnp.arange(n, dtype=edge_index.dtype)
    ei = jnp.concatenate([edge_index, jnp.stack([loop, loop])], axis=1)
    src, dst = ei[0], ei[1]
    h = _gat_layer(x, src, dst, W1, a_src1, a_dst1, b1, HEADS, 256, True)
    h = jax.nn.elu(h)
    out = _gat_layer(h, src, dst, W2, a_src2, a_dst2, b2, 1, 256, False)
    return out


# speedup vs baseline: 4.6571x; 4.1652x over previous
import functools

import jax
import jax.numpy as jnp
from jax import lax
from jax.experimental import pallas as pl
from jax.experimental.pallas import tpu as pltpu
from jax.experimental.pallas import tpu_sc as plsc

N = 10000
E = 160000
HEADS = 4

NC, NS, L = 2, 16, 16
KB = 128
E2 = -(-E // (NS * KB)) * NS * KB


def _mm_kernel(x_ref, w_ref, o_ref):
    o_ref[...] = jnp.dot(x_ref[...], w_ref[...],
                         preferred_element_type=jnp.float32)


def _matmul(x, w, blk_r=1024):
    n, k = x.shape
    _, m = w.shape
    n_pad = ((n + blk_r - 1) // blk_r) * blk_r
    if n_pad != n:
        x = jnp.pad(x, ((0, n_pad - n), (0, 0)))
    out = pl.pallas_call(
        _mm_kernel,
        grid=(n_pad // blk_r,),
        in_specs=[
            pl.BlockSpec((blk_r, k), lambda i: (i, 0)),
            pl.BlockSpec((k, m), lambda i: (0, 0)),
        ],
        out_specs=pl.BlockSpec((blk_r, m), lambda i: (i, 0)),
        out_shape=jax.ShapeDtypeStruct((n_pad, m), jnp.float32),
    )(x, w)
    return out[:n]


def _build_edge_agg(C):
    CPS = C // NC
    EB = E2 // NS
    NB = EB // KB
    RW = N // NS // 8 * 8
    mesh = plsc.VectorSubcoreMesh(core_axis_name="c", subcore_axis_name="s",
                                  num_cores=NC, num_subcores=NS)

    @functools.partial(
        pl.kernel,
        out_type=jax.ShapeDtypeStruct((C, N, 128), jnp.float32),
        mesh=mesh,
        compiler_params=pltpu.CompilerParams(needs_layout_passes=False),
        scratch_types=[
            pltpu.VMEM_SHARED((N, 128), jnp.float32),
            pltpu.VMEM((NB, KB), jnp.int32),
            pltpu.VMEM((NB, KB), jnp.int32),
            pltpu.VMEM((EB,), jnp.float32),
            pltpu.VMEM((KB, 128), jnp.float32),
        ],
    )
    def agg(hc, src2, dst2, ex2, out, acc_sp, src_v, dst_v, ex_v, gbuf):
        cid = lax.axis_index("c")
        sid = lax.axis_index("s")
        wbase = sid * RW
        last = sid == NS - 1

        pltpu.sync_copy(src2.at[sid], src_v)
        pltpu.sync_copy(dst2.at[sid], dst_v)

        for j in range(CPS):
            chunk = cid * CPS + j
            hd = chunk // 2

            def zrow(r, _):
                for c16 in range(128 // L):
                    gbuf[r, pl.ds(c16 * L, L)] = jnp.zeros((L,), jnp.float32)
                return 0
            lax.fori_loop(0, KB, zrow, 0)
            for k in range(RW // KB):
                pltpu.sync_copy(gbuf, acc_sp.at[pl.ds(wbase + k * KB, KB)])
            pltpu.sync_copy(gbuf.at[pl.ds(0, RW % KB)],
                            acc_sp.at[pl.ds(wbase + RW - RW % KB, RW % KB)])
            @pl.when(last)
            def _():
                pltpu.sync_copy(gbuf.at[pl.ds(0, N - RW * NS)],
                                acc_sp.at[pl.ds(RW * NS, N - RW * NS)])

            exoff = pl.multiple_of(hd * E2 + sid * EB, 8)
            pltpu.sync_copy(ex2.at[pl.ds(exoff, EB)], ex_v)
            plsc.subcore_barrier()

            def batch(b, _):
                sidx = src_v.at[b]
                didx = dst_v.at[b]
                pltpu.sync_copy(hc.at[chunk].at[sidx], gbuf)

                def row(r, carry):
                    w = plsc.load_gather(
                        ex_v, [jnp.full((L,), b * KB + r, jnp.int32)])
                    for c16 in range(128 // L):
                        gbuf[r, pl.ds(c16 * L, L)] = (
                            gbuf[r, pl.ds(c16 * L, L)] * w)
                    return carry
                lax.fori_loop(0, KB, row, 0)

                pltpu.sync_copy(gbuf, acc_sp.at[didx], add=True)
                return 0
            lax.fori_loop(0, NB, batch, 0)

            plsc.subcore_barrier()

            for k in range(RW // KB):
                pltpu.sync_copy(acc_sp.at[pl.ds(wbase + k * KB, KB)],
                                out.at[chunk, pl.ds(wbase + k * KB, KB)])
            pltpu.sync_copy(
                acc_sp.at[pl.ds(wbase + RW - RW % KB, RW % KB)],
                out.at[chunk, pl.ds(wbase + RW - RW % KB, RW % KB)])
            @pl.when(last)
            def _():
                pltpu.sync_copy(acc_sp.at[pl.ds(RW * NS, N - RW * NS)],
                                out.at[chunk, pl.ds(RW * NS, N - RW * NS)])

    return agg


_edge_agg = {c: _build_edge_agg(c) for c in (8, 2)}


def _gat_layer(x, src, dst, dst_all, W, a_src, a_dst, bias, heads, concat):
    n = x.shape[0]
    out_c = a_src.shape[1]
    nchunks = heads * out_c // 128
    h = _matmul(x, W)
    hr = h.reshape(n, heads, out_c)
    alpha_src = (hr * a_src[None, :, :]).sum(-1)
    alpha_dst = (hr * a_dst[None, :, :]).sum(-1)
    a_s = jnp.concatenate([alpha_src[src], alpha_src], axis=0)
    a_d = jnp.concatenate([alpha_dst[dst], alpha_dst], axis=0)
    alpha = jax.nn.leaky_relu(a_s + a_d, negative_slope=0.2)
    ex = jnp.exp(alpha)
    denom = jax.ops.segment_sum(ex, dst_all, num_segments=n)

    hcT = h.reshape(n, nchunks, 128).transpose(1, 0, 2)
    nb = E2 // NS // KB
    pad = E2 - E
    exw = jnp.pad(ex[:E].T, ((0, 0), (0, pad))).reshape(heads * E2)
    src2 = jnp.pad(src, (0, pad)).reshape(NS, nb, KB)
    dst2 = jnp.pad(dst, (0, pad)).reshape(NS, nb, KB)
    acc = _edge_agg[nchunks](hcT, src2, dst2, exw)
    acch = acc.transpose(1, 0, 2).reshape(n, heads * out_c)

    ex_self = ex[E:]
    rep = heads * out_c // heads
    out = acch + jnp.repeat(ex_self, rep, axis=1) * h
    out = out / (jnp.repeat(denom, rep, axis=1) + 1e-16)
    if not concat:
        out = out.reshape(n, heads, out_c).mean(axis=1)
    return out + bias


def kernel(x, edge_index, W1, a_src1, a_dst1, b1, W2, a_src2, a_dst2, b2):
    n = x.shape[0]
    src, dst = edge_index[0], edge_index[1]
    loop = jnp.arange(n, dtype=edge_index.dtype)
    dst_all = jnp.concatenate([dst, loop])
    h = _gat_layer(x, src, dst, dst_all, W1, a_src1, a_dst1, b1, HEADS, True)
    h = jax.nn.elu(h)
    out = _gat_layer(h, src, dst, dst_all, W2, a_src2, a_dst2, b2, 1, False)
    return out


# speedup vs baseline: 11.6795x; 2.5079x over previous
import functools

import jax
import jax.numpy as jnp
from jax import lax
from jax.experimental import pallas as pl
from jax.experimental.pallas import tpu as pltpu
from jax.experimental.pallas import tpu_sc as plsc

N = 10000
E = 160000
HEADS = 4

NC, NS, L = 2, 16, 16
KB = 128
E2 = -(-E // (NS * KB)) * NS * KB


def _mm_kernel(x_ref, w_ref, o_ref):
    o_ref[...] = jnp.dot(x_ref[...], w_ref[...],
                         preferred_element_type=jnp.float32)


def _matmul(x, w, blk_r=1024):
    n, k = x.shape
    _, m = w.shape
    n_pad = ((n + blk_r - 1) // blk_r) * blk_r
    if n_pad != n:
        x = jnp.pad(x, ((0, n_pad - n), (0, 0)))
    out = pl.pallas_call(
        _mm_kernel,
        grid=(n_pad // blk_r,),
        in_specs=[
            pl.BlockSpec((blk_r, k), lambda i: (i, 0)),
            pl.BlockSpec((k, m), lambda i: (0, 0)),
        ],
        out_specs=pl.BlockSpec((blk_r, m), lambda i: (i, 0)),
        out_shape=jax.ShapeDtypeStruct((n_pad, m), jnp.float32),
    )(x, w)
    return out[:n]


def _build_edge_agg(C):
    CPS = C // NC
    EB = E2 // NS
    NB = EB // KB
    RW = N // NS // 8 * 8
    mesh = plsc.VectorSubcoreMesh(core_axis_name="c", subcore_axis_name="s",
                                  num_cores=NC, num_subcores=NS)

    @functools.partial(
        pl.kernel,
        out_type=jax.ShapeDtypeStruct((C, N, 128), jnp.float32),
        mesh=mesh,
        compiler_params=pltpu.CompilerParams(needs_layout_passes=False),
        scratch_types=[
            pltpu.VMEM_SHARED((N, 128), jnp.float32),
            pltpu.VMEM((NB, KB), jnp.int32),
            pltpu.VMEM((NB, KB), jnp.int32),
            pltpu.VMEM((EB,), jnp.float32),
            pltpu.VMEM((KB, 128), jnp.float32),
        ],
    )
    def agg(hc, src2, dst2, ex2, out, acc_sp, src_v, dst_v, ex_v, gbuf):
        cid = lax.axis_index("c")
        sid = lax.axis_index("s")
        wbase = sid * RW
        last = sid == NS - 1

        pltpu.sync_copy(src2.at[sid], src_v)
        pltpu.sync_copy(dst2.at[sid], dst_v)

        for j in range(CPS):
            chunk = cid * CPS + j
            hd = chunk // 2

            def zrow(r, _):
                for c16 in range(128 // L):
                    gbuf[r, pl.ds(c16 * L, L)] = jnp.zeros((L,), jnp.float32)
                return 0
            lax.fori_loop(0, KB, zrow, 0)
            for k in range(RW // KB):
                pltpu.sync_copy(gbuf, acc_sp.at[pl.ds(wbase + k * KB, KB)])
            pltpu.sync_copy(gbuf.at[pl.ds(0, RW % KB)],
                            acc_sp.at[pl.ds(wbase + RW - RW % KB, RW % KB)])
            @pl.when(last)
            def _():
                pltpu.sync_copy(gbuf.at[pl.ds(0, N - RW * NS)],
                                acc_sp.at[pl.ds(RW * NS, N - RW * NS)])

            exoff = pl.multiple_of(hd * E2 + sid * EB, 8)
            pltpu.sync_copy(ex2.at[pl.ds(exoff, EB)], ex_v)
            plsc.subcore_barrier()

            def batch(b, _):
                sidx = src_v.at[b]
                didx = dst_v.at[b]
                pltpu.sync_copy(hc.at[chunk].at[sidx], gbuf)

                def row(r, carry):
                    w = plsc.load_gather(
                        ex_v, [jnp.full((L,), b * KB + r, jnp.int32)])
                    for c16 in range(128 // L):
                        gbuf[r, pl.ds(c16 * L, L)] = (
                            gbuf[r, pl.ds(c16 * L, L)] * w)
                    return carry
                lax.fori_loop(0, KB, row, 0)

                pltpu.sync_copy(gbuf, acc_sp.at[didx], add=True)
                return 0
            lax.fori_loop(0, NB, batch, 0)

            plsc.subcore_barrier()

            for k in range(RW // KB):
                pltpu.sync_copy(acc_sp.at[pl.ds(wbase + k * KB, KB)],
                                out.at[chunk, pl.ds(wbase + k * KB, KB)])
            pltpu.sync_copy(
                acc_sp.at[pl.ds(wbase + RW - RW % KB, RW % KB)],
                out.at[chunk, pl.ds(wbase + RW - RW % KB, RW % KB)])
            @pl.when(last)
            def _():
                pltpu.sync_copy(acc_sp.at[pl.ds(RW * NS, N - RW * NS)],
                                out.at[chunk, pl.ds(RW * NS, N - RW * NS)])

    return agg


_edge_agg = {c: _build_edge_agg(c) for c in (8, 2)}


def _build_edge_attn(H):
    HPS = max(H // NC, 1)
    EB = E2 // NS
    G = EB // L
    mesh = plsc.VectorSubcoreMesh(core_axis_name="c", subcore_axis_name="s",
                                  num_cores=NC, num_subcores=NS)
    scr = ([pltpu.VMEM((N,), jnp.float32)] * (3 * HPS)
           + [pltpu.VMEM((EB,), jnp.int32)] * 2
           + [pltpu.VMEM((EB,), jnp.float32)] * HPS)

    @functools.partial(
        pl.kernel,
        out_type=(jax.ShapeDtypeStruct((H * E2,), jnp.float32),
                  jax.ShapeDtypeStruct((H * NS * N,), jnp.float32)),
        mesh=mesh,
        compiler_params=pltpu.CompilerParams(needs_layout_passes=False),
        scratch_types=scr,
    )
    def attn(asrc_f, adst_f, srcf, dstf, exw, denom_f, *scratch):
        as_t = scratch[0:HPS]
        ad_t = scratch[HPS:2 * HPS]
        den = scratch[2 * HPS:3 * HPS]
        src_v, dst_v = scratch[3 * HPS], scratch[3 * HPS + 1]
        exb = scratch[3 * HPS + 2:]
        cid = lax.axis_index("c")
        sid = lax.axis_index("s")

        for j in range(HPS):
            h = cid * HPS + j if H > 1 else 0
            off = pl.multiple_of(h * N, 8)
            pltpu.sync_copy(asrc_f.at[pl.ds(off, N)], as_t[j])
            pltpu.sync_copy(adst_f.at[pl.ds(off, N)], ad_t[j])
        ebase = sid * EB
        pltpu.sync_copy(srcf.at[pl.ds(ebase, EB)], src_v)
        pltpu.sync_copy(dstf.at[pl.ds(ebase, EB)], dst_v)

        def zero(i, _):
            for j in range(HPS):
                den[j][pl.ds(i * L, L)] = jnp.zeros((L,), jnp.float32)
            return 0
        lax.fori_loop(0, N // L, zero, 0)

        def grp(g, _):
            sv = src_v[pl.ds(g * L, L)]
            dv = dst_v[pl.ds(g * L, L)]
            ge = ebase + g * L + lax.iota(jnp.int32, L)
            valid = ge < E
            for j in range(HPS):
                a = plsc.load_gather(as_t[j], [sv])
                b = plsc.load_gather(ad_t[j], [dv])
                al = a + b
                al = jnp.where(al > 0, al, al * jnp.float32(0.2))
                e = jnp.where(valid, jnp.exp(al), jnp.float32(0.0))
                exb[j][pl.ds(g * L, L)] = e
                plsc.addupdate_scatter(den[j], [dv], e)
            return 0
        lax.fori_loop(0, G, grp, 0)

        def write():
            for j in range(HPS):
                h = cid * HPS + j if H > 1 else 0
                off = pl.multiple_of(h * E2 + ebase, 8)
                pltpu.sync_copy(exb[j], exw.at[pl.ds(off, EB)])
                doff = pl.multiple_of((h * NS + sid) * N, 8)
                pltpu.sync_copy(den[j], denom_f.at[pl.ds(doff, N)])
        if H == 1:
            pl.when(cid == 0)(write)
        else:
            write()

    return attn


_edge_attn = {h: _build_edge_attn(h) for h in (4, 1)}


def _gat_layer(x, srcf, dstf, src2, dst2, W, a_src, a_dst, bias, heads,
               concat):
    n = x.shape[0]
    out_c = a_src.shape[1]
    nchunks = heads * out_c // 128
    h = _matmul(x, W)
    hr = h.reshape(n, heads, out_c)
    alpha_src = (hr * a_src[None, :, :]).sum(-1)
    alpha_dst = (hr * a_dst[None, :, :]).sum(-1)

    exw, denom_f = _edge_attn[heads](
        alpha_src.T.reshape(heads * n), alpha_dst.T.reshape(heads * n),
        srcf, dstf)
    denom = denom_f.reshape(heads, NS, n).sum(axis=1).T
    ex_self = jnp.exp(jax.nn.leaky_relu(alpha_src + alpha_dst,
                                        negative_slope=0.2))

    hcT = h.reshape(n, nchunks, 128).transpose(1, 0, 2)
    acc = _edge_agg[nchunks](hcT, src2, dst2, exw)
    acch = acc.transpose(1, 0, 2).reshape(n, heads * out_c)

    rep = out_c
    out = acch + jnp.repeat(ex_self, rep, axis=1) * h
    out = out / (jnp.repeat(denom + ex_self, rep, axis=1) + 1e-16)
    if not concat:
        out = out.reshape(n, heads, out_c).mean(axis=1)
    return out + bias


def kernel(x, edge_index, W1, a_src1, a_dst1, b1, W2, a_src2, a_dst2, b2):
    src, dst = edge_index[0], edge_index[1]
    pad = E2 - E
    nb = E2 // NS // KB
    srcf = jnp.pad(src, (0, pad))
    dstf = jnp.pad(dst, (0, pad))
    src2 = srcf.reshape(NS, nb, KB)
    dst2 = dstf.reshape(NS, nb, KB)
    h = _gat_layer(x, srcf, dstf, src2, dst2, W1, a_src1, a_dst1, b1,
                   HEADS, True)
    h = jax.nn.elu(h)
    out = _gat_layer(h, srcf, dstf, src2, dst2, W2, a_src2, a_dst2, b2,
                     1, False)
    return out


# speedup vs baseline: 12.6931x; 1.0868x over previous
import functools

import jax
import jax.numpy as jnp
from jax import lax
from jax.experimental import pallas as pl
from jax.experimental.pallas import tpu as pltpu
from jax.experimental.pallas import tpu_sc as plsc

N = 10000
E = 160000
HEADS = 4

NC, NS, L = 2, 16, 16
KB = 128
E2 = -(-E // (2 * NS * KB)) * 2 * NS * KB


def _mm_kernel(x_ref, w_ref, o_ref):
    o_ref[...] = jnp.dot(x_ref[...], w_ref[...],
                         preferred_element_type=jnp.float32)


def _matmul(x, w, blk_r=1024):
    n, k = x.shape
    _, m = w.shape
    n_pad = ((n + blk_r - 1) // blk_r) * blk_r
    if n_pad != n:
        x = jnp.pad(x, ((0, n_pad - n), (0, 0)))
    out = pl.pallas_call(
        _mm_kernel,
        grid=(n_pad // blk_r,),
        in_specs=[
            pl.BlockSpec((blk_r, k), lambda i: (i, 0)),
            pl.BlockSpec((k, m), lambda i: (0, 0)),
        ],
        out_specs=pl.BlockSpec((blk_r, m), lambda i: (i, 0)),
        out_shape=jax.ShapeDtypeStruct((n_pad, m), jnp.float32),
    )(x, w)
    return out[:n]


def _build_edge_agg(C):
    CPS = C // NC
    EB = E2 // NS
    NB = EB // KB
    NBH = NB // 2
    RW = N // NS // 8 * 8
    mesh = plsc.VectorSubcoreMesh(core_axis_name="c", subcore_axis_name="s",
                                  num_cores=NC, num_subcores=NS)

    @functools.partial(
        pl.kernel,
        out_type=jax.ShapeDtypeStruct((C, N, 128), jnp.float32),
        mesh=mesh,
        compiler_params=pltpu.CompilerParams(needs_layout_passes=False),
        scratch_types=[
            pltpu.VMEM_SHARED((N, 128), jnp.float32),
            pltpu.VMEM((NBH, KB), jnp.int32),
            pltpu.VMEM((NBH, KB), jnp.int32),
            pltpu.VMEM((NBH * KB,), jnp.float32),
            pltpu.VMEM((KB, 128), jnp.float32),
            pltpu.VMEM((KB, 128), jnp.float32),
            pltpu.SemaphoreType.DMA,
            pltpu.SemaphoreType.DMA,
        ],
    )
    def agg(hc, src2, dst2, ex2, out, acc_sp, src_v, dst_v, ex_v,
            g0, g1, sem0, sem1):
        cid = lax.axis_index("c")
        sid = lax.axis_index("s")
        wbase = sid * RW
        last = sid == NS - 1

        for j in range(CPS):
            chunk = cid * CPS + j
            hd = chunk // 2

            @plsc.parallel_loop(0, KB, unroll=4)
            def zrow(r):
                for c16 in range(128 // L):
                    g0[r, pl.ds(c16 * L, L)] = jnp.zeros((L,), jnp.float32)
            for k in range(RW // KB):
                pltpu.sync_copy(g0, acc_sp.at[pl.ds(wbase + k * KB, KB)])
            pltpu.sync_copy(g0.at[pl.ds(0, RW % KB)],
                            acc_sp.at[pl.ds(wbase + RW - RW % KB, RW % KB)])
            @pl.when(last)
            def _():
                pltpu.sync_copy(g0.at[pl.ds(0, N - RW * NS)],
                                acc_sp.at[pl.ds(RW * NS, N - RW * NS)])
            plsc.subcore_barrier()

            for half in range(2):
                pltpu.sync_copy(src2.at[sid, pl.ds(half * NBH, NBH)], src_v)
                pltpu.sync_copy(dst2.at[sid, pl.ds(half * NBH, NBH)], dst_v)
                exoff = pl.multiple_of(
                    hd * E2 + sid * EB + half * NBH * KB, 8)
                pltpu.sync_copy(ex2.at[pl.ds(exoff, NBH * KB)], ex_v)

                pltpu.async_copy(hc.at[chunk].at[src_v.at[0]], g0, sem0)
                pltpu.async_copy(hc.at[chunk].at[src_v.at[1]], g1, sem1)

                def pair(ph, _):
                    more = ph < NBH // 2 - 1
                    for par, (g, sem) in enumerate(((g0, sem0),
                                                    (g1, sem1))):
                        b = 2 * ph + par
                        pltpu.make_async_copy(
                            hc.at[chunk].at[src_v.at[b]], g, sem).wait()

                        @plsc.parallel_loop(0, KB, unroll=4)
                        def mrow(r):
                            w = plsc.load_gather(
                                ex_v,
                                [jnp.full((L,), b * KB + r, jnp.int32)])
                            for c16 in range(128 // L):
                                g[r, pl.ds(c16 * L, L)] = (
                                    g[r, pl.ds(c16 * L, L)] * w)

                        pltpu.sync_copy(g, acc_sp.at[dst_v.at[b]], add=True)

                        @pl.when(more)
                        def _():
                            pltpu.async_copy(
                                hc.at[chunk].at[src_v.at[b + 2]], g, sem)
                    return 0
                lax.fori_loop(0, NBH // 2, pair, 0)

            plsc.subcore_barrier()

            for k in range(RW // KB):
                pltpu.sync_copy(acc_sp.at[pl.ds(wbase + k * KB, KB)],
                                out.at[chunk, pl.ds(wbase + k * KB, KB)])
            pltpu.sync_copy(
                acc_sp.at[pl.ds(wbase + RW - RW % KB, RW % KB)],
                out.at[chunk, pl.ds(wbase + RW - RW % KB, RW % KB)])
            @pl.when(last)
            def _():
                pltpu.sync_copy(acc_sp.at[pl.ds(RW * NS, N - RW * NS)],
                                out.at[chunk, pl.ds(RW * NS, N - RW * NS)])

    return agg


_edge_agg = {c: _build_edge_agg(c) for c in (8, 2)}


def _build_edge_attn(H):
    HPS = max(H // NC, 1)
    EB = E2 // NS
    G = EB // L
    mesh = plsc.VectorSubcoreMesh(core_axis_name="c", subcore_axis_name="s",
                                  num_cores=NC, num_subcores=NS)
    scr = ([pltpu.VMEM((N,), jnp.float32)] * (3 * HPS)
           + [pltpu.VMEM((EB,), jnp.int32)] * 2
           + [pltpu.VMEM((EB,), jnp.float32)] * HPS)

    @functools.partial(
        pl.kernel,
        out_type=(jax.ShapeDtypeStruct((H * E2,), jnp.float32),
                  jax.ShapeDtypeStruct((H * NS * N,), jnp.float32)),
        mesh=mesh,
        compiler_params=pltpu.CompilerParams(needs_layout_passes=False),
        scratch_types=scr,
    )
    def attn(asrc_f, adst_f, srcf, dstf, exw, denom_f, *scratch):
        as_t = scratch[0:HPS]
        ad_t = scratch[HPS:2 * HPS]
        den = scratch[2 * HPS:3 * HPS]
        src_v, dst_v = scratch[3 * HPS], scratch[3 * HPS + 1]
        exb = scratch[3 * HPS + 2:]
        cid = lax.axis_index("c")
        sid = lax.axis_index("s")

        for j in range(HPS):
            h = cid * HPS + j if H > 1 else 0
            off = pl.multiple_of(h * N, 8)
            pltpu.sync_copy(asrc_f.at[pl.ds(off, N)], as_t[j])
            pltpu.sync_copy(adst_f.at[pl.ds(off, N)], ad_t[j])
        ebase = sid * EB
        pltpu.sync_copy(srcf.at[pl.ds(ebase, EB)], src_v)
        pltpu.sync_copy(dstf.at[pl.ds(ebase, EB)], dst_v)

        def zero(i, _):
            for j in range(HPS):
                den[j][pl.ds(i * L, L)] = jnp.zeros((L,), jnp.float32)
            return 0
        lax.fori_loop(0, N // L, zero, 0)

        def grp(g, _):
            sv = src_v[pl.ds(g * L, L)]
            dv = dst_v[pl.ds(g * L, L)]
            ge = ebase + g * L + lax.iota(jnp.int32, L)
            valid = ge < E
            for j in range(HPS):
                a = plsc.load_gather(as_t[j], [sv])
                b = plsc.load_gather(ad_t[j], [dv])
                al = a + b
                al = jnp.where(al > 0, al, al * jnp.float32(0.2))
                e = jnp.where(valid, jnp.exp(al), jnp.float32(0.0))
                exb[j][pl.ds(g * L, L)] = e
                plsc.addupdate_scatter(den[j], [dv], e)
            return 0
        lax.fori_loop(0, G, grp, 0)

        def write():
            for j in range(HPS):
                h = cid * HPS + j if H > 1 else 0
                off = pl.multiple_of(h * E2 + ebase, 8)
                pltpu.sync_copy(exb[j], exw.at[pl.ds(off, EB)])
                doff = pl.multiple_of((h * NS + sid) * N, 8)
                pltpu.sync_copy(den[j], denom_f.at[pl.ds(doff, N)])
        if H == 1:
            pl.when(cid == 0)(write)
        else:
            write()

    return attn


_edge_attn = {h: _build_edge_attn(h) for h in (4, 1)}


def _gat_layer(x, srcf, dstf, src2, dst2, W, a_src, a_dst, bias, heads,
               concat):
    n = x.shape[0]
    out_c = a_src.shape[1]
    nchunks = heads * out_c // 128
    h = _matmul(x, W)
    hr = h.reshape(n, heads, out_c)
    alpha_src = (hr * a_src[None, :, :]).sum(-1)
    alpha_dst = (hr * a_dst[None, :, :]).sum(-1)

    exw, denom_f = _edge_attn[heads](
        alpha_src.T.reshape(heads * n), alpha_dst.T.reshape(heads * n),
        srcf, dstf)
    denom = denom_f.reshape(heads, NS, n).sum(axis=1).T
    ex_self = jnp.exp(jax.nn.leaky_relu(alpha_src + alpha_dst,
                                        negative_slope=0.2))

    hcT = h.reshape(n, nchunks, 128).transpose(1, 0, 2)
    acc = _edge_agg[nchunks](hcT, src2, dst2, exw)
    acch = acc.transpose(1, 0, 2).reshape(n, heads * out_c)

    rep = out_c
    out = acch + jnp.repeat(ex_self, rep, axis=1) * h
    out = out / (jnp.repeat(denom + ex_self, rep, axis=1) + 1e-16)
    if not concat:
        out = out.reshape(n, heads, out_c).mean(axis=1)
    return out + bias


def kernel(x, edge_index, W1, a_src1, a_dst1, b1, W2, a_src2, a_dst2, b2):
    src, dst = edge_index[0], edge_index[1]
    pad = E2 - E
    nb = E2 // NS // KB
    srcf = jnp.pad(src, (0, pad))
    dstf = jnp.pad(dst, (0, pad))
    src2 = srcf.reshape(NS, nb, KB)
    dst2 = dstf.reshape(NS, nb, KB)
    h = _gat_layer(x, srcf, dstf, src2, dst2, W1, a_src1, a_dst1, b1,
                   HEADS, True)
    h = jax.nn.elu(h)
    out = _gat_layer(h, srcf, dstf, src2, dst2, W2, a_src2, a_dst2, b2,
                     1, False)
    return out


# speedup vs baseline: 15.7930x; 1.2442x over previous
import functools

import jax
import jax.numpy as jnp
from jax import lax
from jax.experimental import pallas as pl
from jax.experimental.pallas import tpu as pltpu
from jax.experimental.pallas import tpu_sc as plsc

N = 10000
E = 160000
HEADS = 4

NC, NS, L = 2, 16, 16
KB = 128
E2 = -(-E // (2 * NS * KB)) * 2 * NS * KB


BLK = 1000


def _leaky(x):
    return jnp.where(x > 0, x, x * jnp.float32(0.2))


def _proj(x, W, a_src, a_dst):
    H, out_c = a_src.shape
    K = x.shape[1]
    C = H * out_c // 128

    def body(x_ref, w_ref, as_ref, ad_ref, hc_ref, al_s_ref, al_d_ref,
             exs_ref):
        h = jnp.dot(x_ref[...], w_ref[...],
                    preferred_element_type=jnp.float32)
        for c in range(C):
            hc_ref[c] = h[:, c * 128:(c + 1) * 128]
        als = []
        ald = []
        for hd in range(H):
            blkh = h[:, hd * out_c:(hd + 1) * out_c]
            als.append((blkh * as_ref[hd][None, :]).sum(-1, keepdims=True))
            ald.append((blkh * ad_ref[hd][None, :]).sum(-1, keepdims=True))
        als = jnp.concatenate(als, axis=1)
        ald = jnp.concatenate(ald, axis=1)
        al_s_ref[...] = als
        al_d_ref[...] = ald
        exs_ref[...] = jnp.exp(_leaky(als + ald))

    return pl.pallas_call(
        body,
        grid=(N // BLK,),
        in_specs=[
            pl.BlockSpec((BLK, K), lambda i: (i, 0)),
            pl.BlockSpec((K, H * out_c), lambda i: (0, 0)),
            pl.BlockSpec((H, out_c), lambda i: (0, 0)),
            pl.BlockSpec((H, out_c), lambda i: (0, 0)),
        ],
        out_specs=[
            pl.BlockSpec((C, BLK, 128), lambda i: (0, i, 0)),
            pl.BlockSpec((BLK, H), lambda i: (i, 0)),
            pl.BlockSpec((BLK, H), lambda i: (i, 0)),
            pl.BlockSpec((BLK, H), lambda i: (i, 0)),
        ],
        out_shape=[
            jax.ShapeDtypeStruct((C, N, 128), jnp.float32),
            jax.ShapeDtypeStruct((N, H), jnp.float32),
            jax.ShapeDtypeStruct((N, H), jnp.float32),
            jax.ShapeDtypeStruct((N, H), jnp.float32),
        ],
    )(x, W, a_src, a_dst)


def _mid(acc, hc, dent, exs, b1, W2, a_src2, a_dst2):
    C1 = acc.shape[0]
    H2, out_c2 = a_src2.shape
    C2 = H2 * out_c2 // 128

    def body(acc_ref, hc_ref, dent_ref, exs_ref, b1_ref, w2_ref, as2_ref,
             ad2_ref, hc2_ref, al_s_ref, al_d_ref, exs2_ref):
        cols = []
        for c in range(C1):
            hd = c // 2
            col = acc_ref[c] + exs_ref[...][:, hd:hd + 1] * hc_ref[c]
            col = col / (dent_ref[...][:, hd:hd + 1] + jnp.float32(1e-16))
            col = col + b1_ref[...][:, c * 128:(c + 1) * 128]
            cols.append(col)
        hrow = jnp.concatenate(cols, axis=1)
        hrow = jnp.where(hrow > 0, hrow, jnp.exp(hrow) - jnp.float32(1.0))
        h2 = jnp.dot(hrow, w2_ref[...], preferred_element_type=jnp.float32)
        for c in range(C2):
            hc2_ref[c] = h2[:, c * 128:(c + 1) * 128]
        als = []
        ald = []
        for hd in range(H2):
            blkh = h2[:, hd * out_c2:(hd + 1) * out_c2]
            als.append((blkh * as2_ref[hd][None, :]).sum(-1, keepdims=True))
            ald.append((blkh * ad2_ref[hd][None, :]).sum(-1, keepdims=True))
        als = jnp.concatenate(als, axis=1) if H2 > 1 else als[0]
        ald = jnp.concatenate(ald, axis=1) if H2 > 1 else ald[0]
        al_s_ref[...] = als
        al_d_ref[...] = ald
        exs2_ref[...] = jnp.exp(_leaky(als + ald))

    return pl.pallas_call(
        body,
        grid=(N // BLK,),
        in_specs=[
            pl.BlockSpec((C1, BLK, 128), lambda i: (0, i, 0)),
            pl.BlockSpec((C1, BLK, 128), lambda i: (0, i, 0)),
            pl.BlockSpec((BLK, C1 // 2), lambda i: (i, 0)),
            pl.BlockSpec((BLK, C1 // 2), lambda i: (i, 0)),
            pl.BlockSpec((1, C1 * 128), lambda i: (0, 0)),
            pl.BlockSpec((C1 * 128, H2 * out_c2), lambda i: (0, 0)),
            pl.BlockSpec((H2, out_c2), lambda i: (0, 0)),
            pl.BlockSpec((H2, out_c2), lambda i: (0, 0)),
        ],
        out_specs=[
            pl.BlockSpec((C2, BLK, 128), lambda i: (0, i, 0)),
            pl.BlockSpec((BLK, H2), lambda i: (i, 0)),
            pl.BlockSpec((BLK, H2), lambda i: (i, 0)),
            pl.BlockSpec((BLK, H2), lambda i: (i, 0)),
        ],
        out_shape=[
            jax.ShapeDtypeStruct((C2, N, 128), jnp.float32),
            jax.ShapeDtypeStruct((N, H2), jnp.float32),
            jax.ShapeDtypeStruct((N, H2), jnp.float32),
            jax.ShapeDtypeStruct((N, H2), jnp.float32),
        ],
    )(acc, hc, dent, exs, b1.reshape(1, -1), W2, a_src2, a_dst2)


def _final(acc2, hc2, dent2, exs2, b2):
    C2 = acc2.shape[0]

    def body(acc_ref, hc_ref, dent_ref, exs_ref, b2_ref, o_ref):
        cols = []
        for c in range(C2):
            col = acc_ref[c] + exs_ref[...] * hc_ref[c]
            col = col / (dent_ref[...] + jnp.float32(1e-16))
            cols.append(col + b2_ref[...][:, c * 128:(c + 1) * 128])
        o_ref[...] = jnp.concatenate(cols, axis=1)

    return pl.pallas_call(
        body,
        grid=(N // BLK,),
        in_specs=[
            pl.BlockSpec((C2, BLK, 128), lambda i: (0, i, 0)),
            pl.BlockSpec((C2, BLK, 128), lambda i: (0, i, 0)),
            pl.BlockSpec((BLK, 1), lambda i: (i, 0)),
            pl.BlockSpec((BLK, 1), lambda i: (i, 0)),
            pl.BlockSpec((1, C2 * 128), lambda i: (0, 0)),
        ],
        out_specs=pl.BlockSpec((BLK, C2 * 128), lambda i: (i, 0)),
        out_shape=jax.ShapeDtypeStruct((N, C2 * 128), jnp.float32),
    )(acc2, hc2, dent2, exs2, b2.reshape(1, -1))


def _build_edge_agg(C):
    CPS = C // NC
    EB = E2 // NS
    NB = EB // KB
    NBH = NB // 2
    RW = N // NS // 8 * 8
    mesh = plsc.VectorSubcoreMesh(core_axis_name="c", subcore_axis_name="s",
                                  num_cores=NC, num_subcores=NS)

    @functools.partial(
        pl.kernel,
        out_type=jax.ShapeDtypeStruct((C, N, 128), jnp.float32),
        mesh=mesh,
        compiler_params=pltpu.CompilerParams(needs_layout_passes=False),
        scratch_types=[
            pltpu.VMEM_SHARED((N, 128), jnp.float32),
            pltpu.VMEM((NBH, KB), jnp.int32),
            pltpu.VMEM((NBH, KB), jnp.int32),
            pltpu.VMEM((NBH * KB,), jnp.float32),
            pltpu.VMEM((KB, 128), jnp.float32),
            pltpu.VMEM((KB, 128), jnp.float32),
            pltpu.SemaphoreType.DMA,
            pltpu.SemaphoreType.DMA,
        ],
    )
    def agg(hc, src2, dst2, ex2, out, acc_sp, src_v, dst_v, ex_v,
            g0, g1, sem0, sem1):
        cid = lax.axis_index("c")
        sid = lax.axis_index("s")
        wbase = sid * RW
        last = sid == NS - 1

        for j in range(CPS):
            chunk = cid * CPS + j
            hd = chunk // 2

            @plsc.parallel_loop(0, KB, unroll=4)
            def zrow(r):
                for c16 in range(128 // L):
                    g0[r, pl.ds(c16 * L, L)] = jnp.zeros((L,), jnp.float32)
            for k in range(RW // KB):
                pltpu.sync_copy(g0, acc_sp.at[pl.ds(wbase + k * KB, KB)])
            pltpu.sync_copy(g0.at[pl.ds(0, RW % KB)],
                            acc_sp.at[pl.ds(wbase + RW - RW % KB, RW % KB)])
            @pl.when(last)
            def _():
                pltpu.sync_copy(g0.at[pl.ds(0, N - RW * NS)],
                                acc_sp.at[pl.ds(RW * NS, N - RW * NS)])
            plsc.subcore_barrier()

            for half in range(2):
                pltpu.sync_copy(src2.at[sid, pl.ds(half * NBH, NBH)], src_v)
                pltpu.sync_copy(dst2.at[sid, pl.ds(half * NBH, NBH)], dst_v)
                exoff = pl.multiple_of(
                    hd * E2 + sid * EB + half * NBH * KB, 8)
                pltpu.sync_copy(ex2.at[pl.ds(exoff, NBH * KB)], ex_v)

                pltpu.async_copy(hc.at[chunk].at[src_v.at[0]], g0, sem0)
                pltpu.async_copy(hc.at[chunk].at[src_v.at[1]], g1, sem1)

                def pair(ph, _):
                    more = ph < NBH // 2 - 1
                    for par, (g, sem) in enumerate(((g0, sem0),
                                                    (g1, sem1))):
                        b = 2 * ph + par
                        pltpu.make_async_copy(
                            hc.at[chunk].at[src_v.at[b]], g, sem).wait()

                        @plsc.parallel_loop(0, KB, unroll=4)
                        def mrow(r):
                            w = plsc.load_gather(
                                ex_v,
                                [jnp.full((L,), b * KB + r, jnp.int32)])
                            for c16 in range(128 // L):
                                g[r, pl.ds(c16 * L, L)] = (
                                    g[r, pl.ds(c16 * L, L)] * w)

                        pltpu.sync_copy(g, acc_sp.at[dst_v.at[b]], add=True)

                        @pl.when(more)
                        def _():
                            pltpu.async_copy(
                                hc.at[chunk].at[src_v.at[b + 2]], g, sem)
                    return 0
                lax.fori_loop(0, NBH // 2, pair, 0)

            plsc.subcore_barrier()

            for k in range(RW // KB):
                pltpu.sync_copy(acc_sp.at[pl.ds(wbase + k * KB, KB)],
                                out.at[chunk, pl.ds(wbase + k * KB, KB)])
            pltpu.sync_copy(
                acc_sp.at[pl.ds(wbase + RW - RW % KB, RW % KB)],
                out.at[chunk, pl.ds(wbase + RW - RW % KB, RW % KB)])
            @pl.when(last)
            def _():
                pltpu.sync_copy(acc_sp.at[pl.ds(RW * NS, N - RW * NS)],
                                out.at[chunk, pl.ds(RW * NS, N - RW * NS)])

    return agg


_edge_agg = {c: _build_edge_agg(c) for c in (8, 2)}


def _build_edge_attn(H):
    HPS = max(H // NC, 1)
    EB = E2 // NS
    G = EB // L
    mesh = plsc.VectorSubcoreMesh(core_axis_name="c", subcore_axis_name="s",
                                  num_cores=NC, num_subcores=NS)
    scr = ([pltpu.VMEM((N,), jnp.float32)] * (3 * HPS)
           + [pltpu.VMEM((EB,), jnp.int32)] * 2
           + [pltpu.VMEM((EB,), jnp.float32)] * HPS)

    @functools.partial(
        pl.kernel,
        out_type=(jax.ShapeDtypeStruct((H * E2,), jnp.float32),
                  jax.ShapeDtypeStruct((H * NS * N,), jnp.float32)),
        mesh=mesh,
        compiler_params=pltpu.CompilerParams(needs_layout_passes=False),
        scratch_types=scr,
    )
    def attn(asrc_f, adst_f, srcf, dstf, exw, denom_f, *scratch):
        as_t = scratch[0:HPS]
        ad_t = scratch[HPS:2 * HPS]
        den = scratch[2 * HPS:3 * HPS]
        src_v, dst_v = scratch[3 * HPS], scratch[3 * HPS + 1]
        exb = scratch[3 * HPS + 2:]
        cid = lax.axis_index("c")
        sid = lax.axis_index("s")

        for j in range(HPS):
            h = cid * HPS + j if H > 1 else 0
            off = pl.multiple_of(h * N, 8)
            pltpu.sync_copy(asrc_f.at[pl.ds(off, N)], as_t[j])
            pltpu.sync_copy(adst_f.at[pl.ds(off, N)], ad_t[j])
        ebase = sid * EB
        pltpu.sync_copy(srcf.at[pl.ds(ebase, EB)], src_v)
        pltpu.sync_copy(dstf.at[pl.ds(ebase, EB)], dst_v)

        def zero(i, _):
            for j in range(HPS):
                den[j][pl.ds(i * L, L)] = jnp.zeros((L,), jnp.float32)
            return 0
        lax.fori_loop(0, N // L, zero, 0)

        def grp(g, _):
            sv = src_v[pl.ds(g * L, L)]
            dv = dst_v[pl.ds(g * L, L)]
            ge = ebase + g * L + lax.iota(jnp.int32, L)
            valid = ge < E
            for j in range(HPS):
                a = plsc.load_gather(as_t[j], [sv])
                b = plsc.load_gather(ad_t[j], [dv])
                al = a + b
                al = jnp.where(al > 0, al, al * jnp.float32(0.2))
                e = jnp.where(valid, jnp.exp(al), jnp.float32(0.0))
                exb[j][pl.ds(g * L, L)] = e
                plsc.addupdate_scatter(den[j], [dv], e)
            return 0
        lax.fori_loop(0, G, grp, 0)

        def write():
            for j in range(HPS):
                h = cid * HPS + j if H > 1 else 0
                off = pl.multiple_of(h * E2 + ebase, 8)
                pltpu.sync_copy(exb[j], exw.at[pl.ds(off, EB)])
                doff = pl.multiple_of((h * NS + sid) * N, 8)
                pltpu.sync_copy(den[j], denom_f.at[pl.ds(doff, N)])
        if H == 1:
            pl.when(cid == 0)(write)
        else:
            write()

    return attn


_edge_attn = {h: _build_edge_attn(h) for h in (4, 1)}


def _attn_and_agg(hc, als, ald, srcf, dstf, src2, dst2, exs, heads):
    n = als.shape[0]
    exw, denom_f = _edge_attn[heads](
        als.T.reshape(heads * n), ald.T.reshape(heads * n), srcf, dstf)
    acc = _edge_agg[hc.shape[0]](hc, src2, dst2, exw)
    dent = denom_f.reshape(heads, NS, n).sum(axis=1).T + exs
    return acc, dent


def kernel(x, edge_index, W1, a_src1, a_dst1, b1, W2, a_src2, a_dst2, b2):
    src, dst = edge_index[0], edge_index[1]
    pad = E2 - E
    nb = E2 // NS // KB
    srcf = jnp.pad(src, (0, pad))
    dstf = jnp.pad(dst, (0, pad))
    src2 = srcf.reshape(NS, nb, KB)
    dst2 = dstf.reshape(NS, nb, KB)

    hc1, als1, ald1, exs1 = _proj(x, W1, a_src1, a_dst1)
    acc1, dent1 = _attn_and_agg(hc1, als1, ald1, srcf, dstf, src2, dst2,
                                exs1, HEADS)
    hc2, als2, ald2, exs2 = _mid(acc1, hc1, dent1, exs1, b1, W2,
                                 a_src2, a_dst2)
    acc2, dent2 = _attn_and_agg(hc2, als2, ald2, srcf, dstf, src2, dst2,
                                exs2, 1)
    return _final(acc2, hc2, dent2, exs2, b2)
